# baseline (speedup 1.0000x reference)
import functools

import jax
import jax.numpy as jnp
from jax import lax
from jax.experimental import pallas as pl
from jax.experimental.pallas import tpu as pltpu
from jax.experimental.pallas import tpu_sc as plsc

DIM = 64
SCALE = 0.125
NC, NS = 2, 16
NW = NC * NS
LB = 128
CHUNK = 256
TWORDS = 2 * 8 * LB * 8


@functools.lru_cache(maxsize=None)
def _build(bsz: int, seq: int):
  assert bsz % (NW * LB) == 0 or bsz == NW * LB
  assert seq % 8 == 0
  s_tiles = seq // 8
  n_chunks = seq // 2
  B = bsz * seq
  mesh = plsc.VectorSubcoreMesh(core_axis_name="c", subcore_axis_name="s")

  @functools.partial(
      pl.kernel,
      out_type=jax.ShapeDtypeStruct((B * DIM,), jnp.float32),
      mesh=mesh,
      compiler_params=pltpu.CompilerParams(
          use_tc_tiling_on_sc=False, needs_layout_passes=False),
      scratch_types=[
          pltpu.VMEM((CHUNK,), jnp.int32),
          pltpu.VMEM((CHUNK,), jnp.int32),
          pltpu.VMEM((CHUNK, DIM), jnp.float32),
          pltpu.VMEM((CHUNK, DIM), jnp.float32),
          pltpu.VMEM((TWORDS,), jnp.float32),
          pltpu.VMEM((TWORDS,), jnp.float32),
          pltpu.SemaphoreType.DMA,
          pltpu.SemaphoreType.DMA,
          pltpu.SemaphoreType.DMA,
          pltpu.SemaphoreType.DMA,
          pltpu.SemaphoreType.DMA,
          pltpu.SemaphoreType.DMA,
      ],
  )
  def embed(ids_hbm, w_hbm, out_hbm, idx0, idx1, r0, r1, t0, t1,
            sem_i0, sem_i1, sem_g0, sem_g1, sem_o0, sem_o1):
    c = lax.axis_index("s") * NC + lax.axis_index("c")

    iota = lax.iota(jnp.int32, 16)
    d128 = [iota * LB + g * 16 * LB for g in range(DIM // 16)]

    slots = (
        (idx0, r0, t0, sem_i0, sem_g0, sem_o0),
        (idx1, r1, t1, sem_i1, sem_g1, sem_o1),
    )

    def idx_off(k):
      return ((k >> 2) * NW + c) * 1024 + (k & 3) * CHUNK

    for b, (idxb, rb, tb, sem_ib, sem_gb, sem_ob) in enumerate(slots):
      pltpu.sync_copy(ids_hbm.at[pl.ds(idx_off(b), CHUNK)], idxb)
      pltpu.async_copy(w_hbm.at[idxb], rb, sem_gb)

    @pl.loop(0, n_chunks // 2)
    def _(i):
      for b, (idxb, rb, tb, sem_ib, sem_gb, sem_ob) in enumerate(slots):
        k = i * 2 + b
        pltpu.make_async_copy(w_hbm.at[idxb], rb, sem_gb).wait()

        @pl.when(k < n_chunks - 2)
        def _():
          pltpu.async_copy(ids_hbm.at[pl.ds(idx_off(k + 2), CHUNK)],
                           idxb, sem_ib)

        @pl.when(k >= 2)
        def _():
          pltpu.make_async_copy(tb, out_hbm.at[pl.ds(0, TWORDS)],
                                sem_ob).wait()

        @pl.loop(0, CHUNK)
        def _(j):
          sbase = (j >> 7) * 8192 + (j & 127)
          for g in range(DIM // 16):
            v = rb[j, pl.ds(g * 16, 16)] * SCALE
            plsc.store_scatter(tb, [d128[g] + sbase], v)

        s0 = (k >> 2) * 8 + (k & 3) * 2
        for t in range(2):
          for d1 in range(8):
            off = (((s0 + t) * 8 + d1) * NW + c) * 1024
            pltpu.async_copy(tb.at[pl.ds(t * 8192 + d1 * 1024, 1024)],
                             out_hbm.at[pl.ds(off, 1024)], sem_ob)

        @pl.when(k < n_chunks - 2)
        def _():
          pltpu.make_async_copy(ids_hbm.at[pl.ds(0, CHUNK)],
                                idxb, sem_ib).wait()
          pltpu.async_copy(w_hbm.at[idxb], rb, sem_gb)

    for b, (idxb, rb, tb, sem_ib, sem_gb, sem_ob) in enumerate(slots):
      pltpu.make_async_copy(tb, out_hbm.at[pl.ds(0, TWORDS)], sem_ob).wait()

  return embed


def kernel(ids, weight):
  bsz, seq = ids.shape
  ids_n = (ids.T.astype(jnp.int32)
           .reshape(seq // 8, 8, bsz // LB, LB)
           .transpose(0, 2, 1, 3)
           .reshape(bsz * seq))
  out1 = _build(bsz, seq)(ids_n, weight)
  return (out1.reshape(seq, 8, bsz // LB, 8, LB)
          .transpose(2, 4, 0, 1, 3)
          .reshape(bsz, seq, DIM))


# speedup vs baseline: 1.5527x; 1.5527x over previous
import functools

import jax
import jax.numpy as jnp
from jax import lax
from jax.experimental import pallas as pl
from jax.experimental.pallas import tpu as pltpu
from jax.experimental.pallas import tpu_sc as plsc

DIM = 64
SCALE = 0.125
NC, NS = 2, 16
NW = NC * NS
LB = 128
CHUNK = 256
TWORDS = 2 * 8 * LB * 8


@functools.lru_cache(maxsize=None)
def _build(bsz: int, seq: int):
  assert bsz % (NW * LB) == 0 or bsz == NW * LB
  assert seq % 8 == 0
  s_tiles = seq // 8
  n_chunks = seq // 2
  B = bsz * seq
  mesh = plsc.VectorSubcoreMesh(core_axis_name="c", subcore_axis_name="s")

  @functools.partial(
      pl.kernel,
      out_type=jax.ShapeDtypeStruct((B * DIM,), jnp.float32),
      mesh=mesh,
      compiler_params=pltpu.CompilerParams(
          use_tc_tiling_on_sc=False, needs_layout_passes=False),
      scratch_types=[
          pltpu.VMEM((CHUNK,), jnp.int32),
          pltpu.VMEM((CHUNK,), jnp.int32),
          pltpu.VMEM((CHUNK, DIM), jnp.float32),
          pltpu.VMEM((CHUNK, DIM), jnp.float32),
          pltpu.VMEM((TWORDS,), jnp.float32),
          pltpu.VMEM((TWORDS,), jnp.float32),
          pltpu.SemaphoreType.DMA,
          pltpu.SemaphoreType.DMA,
          pltpu.SemaphoreType.DMA,
          pltpu.SemaphoreType.DMA,
          pltpu.SemaphoreType.DMA,
          pltpu.SemaphoreType.DMA,
      ],
  )
  def embed(ids_hbm, w_hbm, out_hbm, idx0, idx1, r0, r1, t0, t1,
            sem_i0, sem_i1, sem_g0, sem_g1, sem_o0, sem_o1):
    c = lax.axis_index("s") * NC + lax.axis_index("c")

    iota = lax.iota(jnp.int32, 16)
    d128 = [iota * LB + g * 16 * LB for g in range(DIM // 16)]

    slots = (
        (idx0, r0, t0, sem_i0, sem_g0, sem_o0),
        (idx1, r1, t1, sem_i1, sem_g1, sem_o1),
    )

    def idx_off(k):
      return ((k >> 2) * NW + c) * 1024 + (k & 3) * CHUNK

    for b, (idxb, rb, tb, sem_ib, sem_gb, sem_ob) in enumerate(slots):
      pltpu.sync_copy(ids_hbm.at[pl.ds(idx_off(b), CHUNK)], idxb)
      pltpu.async_copy(w_hbm.at[idxb], rb, sem_gb)

    @pl.loop(0, n_chunks // 2)
    def _(i):
      for b, (idxb, rb, tb, sem_ib, sem_gb, sem_ob) in enumerate(slots):
        k = i * 2 + b
        pltpu.make_async_copy(w_hbm.at[idxb], rb, sem_gb).wait()

        @pl.when(k < n_chunks - 2)
        def _():
          pltpu.async_copy(ids_hbm.at[pl.ds(idx_off(k + 2), CHUNK)],
                           idxb, sem_ib)

        @pl.when(k >= 2)
        def _():
          pltpu.make_async_copy(tb, out_hbm.at[pl.ds(0, TWORDS)],
                                sem_ob).wait()

        @plsc.parallel_loop(0, CHUNK, unroll=8)
        def _(j):
          sbase = (j >> 7) * 8192 + (j & 127)
          for g in range(DIM // 16):
            v = rb[j, pl.ds(g * 16, 16)] * SCALE
            plsc.store_scatter(tb, [d128[g] + sbase], v)

        s0 = (k >> 2) * 8 + (k & 3) * 2
        for t in range(2):
          for d1 in range(8):
            off = (((s0 + t) * 8 + d1) * NW + c) * 1024
            pltpu.async_copy(tb.at[pl.ds(t * 8192 + d1 * 1024, 1024)],
                             out_hbm.at[pl.ds(off, 1024)], sem_ob)

        @pl.when(k < n_chunks - 2)
        def _():
          pltpu.make_async_copy(ids_hbm.at[pl.ds(0, CHUNK)],
                                idxb, sem_ib).wait()
          pltpu.async_copy(w_hbm.at[idxb], rb, sem_gb)

    for b, (idxb, rb, tb, sem_ib, sem_gb, sem_ob) in enumerate(slots):
      pltpu.make_async_copy(tb, out_hbm.at[pl.ds(0, TWORDS)], sem_ob).wait()

  return embed


def kernel(ids, weight):
  bsz, seq = ids.shape
  ids_n = (ids.T.astype(jnp.int32)
           .reshape(seq // 8, 8, bsz // LB, LB)
           .transpose(0, 2, 1, 3)
           .reshape(bsz * seq))
  out1 = _build(bsz, seq)(ids_n, weight)
  return (out1.reshape(seq, 8, bsz // LB, 8, LB)
          .transpose(2, 4, 0, 1, 3)
          .reshape(bsz, seq, DIM))
